# baseline (speedup 1.0000x reference)
import functools

import jax
import jax.numpy as jnp
from jax.experimental import pallas as pl
from jax.experimental.pallas import tpu as pltpu

N_E = 1024
E_DIM = 64
BETA = 0.25

ROWS = 512


def _vq_body(z_ref, emb_ref, zq_ref, idx_ref, loss_ref, perp_ref,
             loss_acc, cnt_acc, *, n_total):
    i = pl.program_id(0)
    nsteps = pl.num_programs(0)

    zb = z_ref[...]
    emb = emb_ref[...]

    e_sq = jnp.sum(emb ** 2, axis=1)
    z_sq = jnp.sum(zb ** 2, axis=1, keepdims=True)
    d = jax.lax.dot_general(zb, emb, (((1,), (1,)), ((), ())),
                            preferred_element_type=jnp.float32)
    dist = (z_sq + e_sq) - 2.0 * d

    min_d = jnp.min(dist, axis=1, keepdims=True)
    lane = jax.lax.broadcasted_iota(jnp.int32, (ROWS, N_E), 1)
    idx = jnp.min(jnp.where(dist == min_d, lane, N_E), axis=1)
    enc = (lane == idx[:, None]).astype(jnp.float32)
    idx_ref[...] = idx[:, None]

    zq = jax.lax.dot_general(enc, emb, (((1,), (0,)), ((), ())),
                             preferred_element_type=jnp.float32)
    diff = zq - zb
    zq_ref[...] = zb + diff

    @pl.when(i == 0)
    def _init():
        loss_acc[0] = 0.0
        cnt_acc[...] = jnp.zeros_like(cnt_acc)

    loss_acc[0] += jnp.sum(diff * diff)
    cnt_acc[...] += jnp.sum(enc, axis=0, keepdims=True)

    @pl.when(i == nsteps - 1)
    def _finalize():
        total = loss_acc[0] / (n_total * E_DIM)
        loss_ref[...] = jnp.full((1, 1), total * (1.0 + BETA), jnp.float32)
        e_mean = cnt_acc[...] / n_total
        ent = e_mean * jnp.log(e_mean + 1e-10)
        perp_ref[...] = jnp.exp(-jnp.sum(ent, axis=1, keepdims=True))


def kernel(z, embedding):
    B, ed, T = z.shape
    n = B * T
    zf = jnp.transpose(z, (0, 2, 1)).reshape(n, ed)
    nsteps = n // ROWS

    zq, idx, loss, perp = pl.pallas_call(
        functools.partial(_vq_body, n_total=n),
        grid=(nsteps,),
        in_specs=[
            pl.BlockSpec((ROWS, ed), lambda i: (i, 0)),
            pl.BlockSpec((N_E, ed), lambda i: (0, 0)),
        ],
        out_specs=[
            pl.BlockSpec((ROWS, ed), lambda i: (i, 0)),
            pl.BlockSpec((ROWS, 1), lambda i: (i, 0)),
            pl.BlockSpec((1, 1), lambda i: (0, 0)),
            pl.BlockSpec((1, 1), lambda i: (0, 0)),
        ],
        out_shape=[
            jax.ShapeDtypeStruct((n, ed), jnp.float32),
            jax.ShapeDtypeStruct((n, 1), jnp.int32),
            jax.ShapeDtypeStruct((1, 1), jnp.float32),
            jax.ShapeDtypeStruct((1, 1), jnp.float32),
        ],
        scratch_shapes=[
            pltpu.SMEM((1,), jnp.float32),
            pltpu.VMEM((1, N_E), jnp.float32),
        ],
    )(zf, embedding)

    z_q_out = jnp.transpose(zq.reshape(B, T, ed), (0, 2, 1))
    enc = jnp.zeros((n, N_E), jnp.float32)
    return loss[0, 0], z_q_out, perp[0, 0], enc, idx


# speedup vs baseline: 1.2604x; 1.2604x over previous
import functools

import jax
import jax.numpy as jnp
from jax.experimental import pallas as pl
from jax.experimental.pallas import tpu as pltpu

N_E = 1024
E_DIM = 64
BETA = 0.25

ROWS = 512


def _vq_body(z_ref, emb_ref, enc_ref, zq_ref, idx_ref, loss_ref, perp_ref,
             loss_acc, cnt_acc, *, n_total):
    i = pl.program_id(0)
    nsteps = pl.num_programs(0)

    zb = z_ref[...]
    emb = emb_ref[...]

    e_sq = jnp.sum(emb ** 2, axis=1)
    z_sq = jnp.sum(zb ** 2, axis=1, keepdims=True)
    d = jax.lax.dot_general(zb, emb, (((1,), (1,)), ((), ())),
                            preferred_element_type=jnp.float32)
    dist = (z_sq + e_sq) - 2.0 * d

    min_d = jnp.min(dist, axis=1, keepdims=True)
    lane_f = jax.lax.broadcasted_iota(jnp.int32, (ROWS, N_E), 1).astype(jnp.float32)
    idx_f = jnp.min(jnp.where(dist == min_d, lane_f, jnp.float32(N_E)),
                    axis=1, keepdims=True)
    enc = jnp.where(lane_f == idx_f, 1.0, 0.0)
    enc_ref[...] = enc
    idx_ref[...] = idx_f.astype(jnp.int32)

    zq = jax.lax.dot_general(enc, emb, (((1,), (0,)), ((), ())),
                             preferred_element_type=jnp.float32)
    diff = zq - zb
    zq_ref[...] = zb + diff

    @pl.when(i == 0)
    def _init():
        loss_acc[0] = 0.0
        cnt_acc[...] = jnp.zeros_like(cnt_acc)

    loss_acc[0] += jnp.sum(diff * diff)
    cnt_acc[...] += jnp.sum(enc, axis=0, keepdims=True)

    @pl.when(i == nsteps - 1)
    def _finalize():
        total = loss_acc[0] / (n_total * E_DIM)
        loss_ref[...] = jnp.full((1, 1), total * (1.0 + BETA), jnp.float32)
        e_mean = cnt_acc[...] / n_total
        ent = e_mean * jnp.log(e_mean + 1e-10)
        perp_ref[...] = jnp.exp(-jnp.sum(ent, axis=1, keepdims=True))


def kernel(z, embedding):
    B, ed, T = z.shape
    n = B * T
    zf = jnp.transpose(z, (0, 2, 1)).reshape(n, ed)
    nsteps = n // ROWS

    enc, zq, idx, loss, perp = pl.pallas_call(
        functools.partial(_vq_body, n_total=n),
        grid=(nsteps,),
        in_specs=[
            pl.BlockSpec((ROWS, ed), lambda i: (i, 0)),
            pl.BlockSpec((N_E, ed), lambda i: (0, 0)),
        ],
        out_specs=[
            pl.BlockSpec((ROWS, N_E), lambda i: (i, 0)),
            pl.BlockSpec((ROWS, ed), lambda i: (i, 0)),
            pl.BlockSpec((ROWS, 1), lambda i: (i, 0)),
            pl.BlockSpec((1, 1), lambda i: (0, 0)),
            pl.BlockSpec((1, 1), lambda i: (0, 0)),
        ],
        out_shape=[
            jax.ShapeDtypeStruct((n, N_E), jnp.float32),
            jax.ShapeDtypeStruct((n, ed), jnp.float32),
            jax.ShapeDtypeStruct((n, 1), jnp.int32),
            jax.ShapeDtypeStruct((1, 1), jnp.float32),
            jax.ShapeDtypeStruct((1, 1), jnp.float32),
        ],
        scratch_shapes=[
            pltpu.SMEM((1,), jnp.float32),
            pltpu.VMEM((1, N_E), jnp.float32),
        ],
    )(zf, embedding)

    z_q_out = jnp.transpose(zq.reshape(B, T, ed), (0, 2, 1))
    return loss[0, 0], z_q_out, perp[0, 0], enc, idx
